# Initial kernel scaffold; baseline (speedup 1.0000x reference)
import jax
import jax.numpy as jnp
from jax.experimental import pallas as pl


def kernel(nlist, positions):
    raise NotImplementedError("write your pallas kernel here")


# speedup vs baseline: 13.4247x; 13.4247x over previous
import functools

import numpy as np

import jax
import jax.numpy as jnp
from jax import lax
from jax.experimental import pallas as pl
from jax.experimental.pallas import tpu as pltpu
from jax.experimental.pallas import tpu_sc as plsc

N = 100000
K = 64
M = N * K
NB = 102
NBPAD = 112
R_MIN = 0.0
R_MAX = 10.0


_BR = 2000

_P_HOST = np.zeros((4 * K, K), dtype=np.float32)
for _j in range(K):
    _P_HOST[4 * _j: 4 * _j + 3, _j] = 1.0


def _tc_norm_body(x_ref, p_ref, o_ref):
    v = x_ref[...]
    sq = v * v
    s = lax.dot_general(
        sq, p_ref[...],
        dimension_numbers=(((1,), (0,)), ((), ())),
        precision=lax.Precision.HIGHEST,
        preferred_element_type=jnp.float32,
    )
    o_ref[...] = jnp.sqrt(s)


def _tc_norm(x2, p):
    return pl.pallas_call(
        _tc_norm_body,
        grid=(N // _BR,),
        in_specs=[
            pl.BlockSpec((_BR, 4 * K), lambda i: (i, 0)),
            pl.BlockSpec((4 * K, K), lambda i: (0, 0)),
        ],
        out_specs=pl.BlockSpec((_BR, K), lambda i: (i, 0)),
        out_shape=jax.ShapeDtypeStruct((N, K), jnp.float32),
    )(x2, p)


_NC = 2
_NS = 16
_NW = _NC * _NS
_PER_W = M // _NW
_CHUNK = 10000
_NCHUNK = _PER_W // _CHUNK
_VPC = _CHUNK // 16


def _sc_hist_body(r_hbm, out_hbm, buf, hist):
    wid = lax.axis_index("s") * _NC + lax.axis_index("c")
    lanes = lax.iota(jnp.int32, 16)
    lane_base = lanes * NBPAD
    ones = jnp.ones((16,), jnp.float32)
    zeros = jnp.zeros((16,), jnp.float32)
    for j in range(16 * NBPAD // 16):
        hist[pl.ds(j * 16, 16)] = zeros
    base = wid * _PER_W
    for c in range(_NCHUNK):
        pltpu.sync_copy(r_hbm.at[pl.ds(base + c * _CHUNK, _CHUNK)], buf)

        def body(i, carry):
            v = buf[pl.ds(i * 16, 16)]
            t = (v / 10.0) * 102
            t = jnp.minimum(t, 101.5)
            t = jnp.maximum(t, 0.0)
            b = t.astype(jnp.int32)
            plsc.addupdate_scatter(hist, [lane_base + b], ones)
            return carry

        lax.fori_loop(0, _VPC, body, 0)
    pltpu.sync_copy(hist, out_hbm.at[wid])


def _sc_hist(r_flat):
    mesh = plsc.VectorSubcoreMesh(core_axis_name="c", subcore_axis_name="s")
    f = pl.kernel(
        _sc_hist_body,
        mesh=mesh,
        out_type=jax.ShapeDtypeStruct((_NW, 16 * NBPAD), jnp.float32),
        scratch_types=[
            pltpu.VMEM((_CHUNK,), jnp.float32),
            pltpu.VMEM((16 * NBPAD,), jnp.float32),
        ],
        compiler_params=pltpu.CompilerParams(needs_layout_passes=False),
    )
    return f(r_flat)


def kernel(nlist, positions):
    x2 = nlist.reshape(N, 4 * K)
    r = _tc_norm(x2, jnp.asarray(_P_HOST))
    partials = _sc_hist(r.reshape(M))
    hist = partials.reshape(_NW * 16, NBPAD).sum(0)
    shell_rs = jnp.linspace(R_MIN, R_MAX, 101)
    vols = shell_rs[1:] ** 3 - shell_rs[:-1] ** 3
    return hist[1:NB - 1] / vols


# speedup vs baseline: 40.2248x; 2.9963x over previous
import jax
import jax.numpy as jnp
from jax import lax
from jax.experimental import pallas as pl
from jax.experimental.pallas import tpu as pltpu
from jax.experimental.pallas import tpu_sc as plsc

N = 100000
K = 64
NP = 100096
MP = K * NP
NB = 102
NBPAD = 112
R_MIN = 0.0
R_MAX = 10.0


_BP = 5888
_NBLK = NP // _BP


def _tc_bins_body(x_ref, o_ref):
    i = pl.program_id(0)
    v = x_ref[...]
    sq = v * v
    s = sq[:, 0, :] + sq[:, 1, :] + sq[:, 2, :]
    r = jnp.sqrt(s)
    t = (r / 10.0) * 102
    t = jnp.minimum(t, 101.5)
    t = jnp.maximum(t, 0.0)
    b = t.astype(jnp.int32)
    b = jnp.clip(b, 0, NB - 1)
    col = i * _BP + lax.broadcasted_iota(jnp.int32, (K, _BP), 1)
    b = jnp.where(col < N, b, NB - 1)
    o_ref[...] = b[:, : _BP // 2] | (b[:, _BP // 2:] << 16)


def _tc_bins(t):
    return pl.pallas_call(
        _tc_bins_body,
        grid=(_NBLK,),
        in_specs=[pl.BlockSpec((K, 4, _BP), lambda i: (0, 0, i))],
        out_specs=pl.BlockSpec((K, _BP // 2), lambda i: (0, i)),
        out_shape=jax.ShapeDtypeStruct((K, NP // 2), jnp.int32),
    )(t)


_NC = 2
_NS = 16
_NW = _NC * _NS
_PER_W = MP // 2 // _NW
_CHUNK = 5888
_NCHUNK = _PER_W // _CHUNK
_VPC = _CHUNK // 16


def _sc_hist_body(b_hbm, out_hbm, buf, hist):
    wid = lax.axis_index("s") * _NC + lax.axis_index("c")
    lanes = lax.iota(jnp.int32, 16)
    lane_base = lanes * NBPAD
    hi_base = lane_base + 16 * NBPAD
    ones = jnp.ones((16,), jnp.float32)
    zeros = jnp.zeros((16,), jnp.float32)
    for j in range(2 * 16 * NBPAD // 16):
        hist[pl.ds(j * 16, 16)] = zeros
    base = wid * _PER_W
    for c in range(_NCHUNK):
        pltpu.sync_copy(b_hbm.at[pl.ds(base + c * _CHUNK, _CHUNK)], buf)

        def body(i, carry):
            w = buf[pl.ds(i * 16, 16)]
            lo = w & 0xFFFF
            hi = lax.shift_right_logical(w, 16)
            plsc.addupdate_scatter(hist, [lane_base + lo], ones)
            plsc.addupdate_scatter(hist, [hi_base + hi], ones)
            return carry

        lax.fori_loop(0, _VPC, body, 0)
    pltpu.sync_copy(hist, out_hbm.at[wid])


def _sc_hist(b_flat):
    mesh = plsc.VectorSubcoreMesh(core_axis_name="c", subcore_axis_name="s")
    f = pl.kernel(
        _sc_hist_body,
        mesh=mesh,
        out_type=jax.ShapeDtypeStruct((_NW, 2 * 16 * NBPAD), jnp.float32),
        scratch_types=[
            pltpu.VMEM((_CHUNK,), jnp.int32),
            pltpu.VMEM((2 * 16 * NBPAD,), jnp.float32),
        ],
        compiler_params=pltpu.CompilerParams(needs_layout_passes=False),
    )
    return f(b_flat)


def kernel(nlist, positions):
    t = jnp.transpose(nlist, (1, 2, 0))
    bins = _tc_bins(t)
    partials = _sc_hist(bins.reshape(MP // 2))
    hist = partials.reshape(_NW * 32, NBPAD).sum(0)
    shell_rs = jnp.linspace(R_MIN, R_MAX, 101)
    vols = shell_rs[1:] ** 3 - shell_rs[:-1] ** 3
    return hist[1:NB - 1] / vols


# speedup vs baseline: 49.4283x; 1.2288x over previous
import jax
import jax.numpy as jnp
from jax import lax
from jax.experimental import pallas as pl
from jax.experimental.pallas import tpu as pltpu
from jax.experimental.pallas import tpu_sc as plsc

N = 100000
K = 64
NP = 100096
MP = K * NP
NB = 102
NBPAD = 112
R_MIN = 0.0
R_MAX = 10.0


_BP = 5888
_NBLK = NP // _BP


def _tc_bins_body(x_ref, o_ref):
    i = pl.program_id(0)
    v = x_ref[...]
    sq = v * v
    s = sq[:, 0, :] + sq[:, 1, :] + sq[:, 2, :]
    r = jnp.sqrt(s)
    t = (r / 10.0) * 102
    t = jnp.minimum(t, 101.5)
    t = jnp.maximum(t, 0.0)
    b = t.astype(jnp.int32)
    col = i * _BP + lax.broadcasted_iota(jnp.int32, (K, _BP), 1)
    b = jnp.where(col < N, b, NB - 1)
    o_ref[...] = b[:, : _BP // 2] | (b[:, _BP // 2:] << 16)


def _tc_bins(t):
    return pl.pallas_call(
        _tc_bins_body,
        grid=(_NBLK,),
        in_specs=[pl.BlockSpec((K, 4, _BP), lambda i: (0, 0, i))],
        out_specs=pl.BlockSpec((K, _BP // 2), lambda i: (0, i)),
        out_shape=jax.ShapeDtypeStruct((K, NP // 2), jnp.int32),
    )(t)


_NC = 2
_NS = 16
_NW = _NC * _NS
_ROWS_W = K // _NW * 2
_CHUNK = 2944
_NCHUNK = (NP // 2) // _CHUNK
_NDMA = 2 * _NCHUNK


def _sc_hist_body(b_hbm, out_hbm, buf0, buf1, hist, sem0, sem1):
    wid = lax.axis_index("s") * _NC + lax.axis_index("c")
    lanes = lax.iota(jnp.int32, 16)
    lane_base = lanes * NBPAD
    hi_base = lane_base + 16 * NBPAD
    ones = jnp.ones((16,), jnp.float32)
    zeros = jnp.zeros((16,), jnp.float32)
    for j in range(2 * 16 * NBPAD // 16):
        hist[pl.ds(j * 16, 16)] = zeros
    bufs = (buf0, buf1)
    sems = (sem0, sem1)
    row0 = wid * 2

    def start(k):
        row = row0 + k // _NCHUNK
        col = (k % _NCHUNK) * _CHUNK
        return pltpu.async_copy(
            b_hbm.at[row, pl.ds(col, _CHUNK)], bufs[k % 2], sems[k % 2])

    pending = start(0)
    for k in range(_NDMA):
        nxt = start(k + 1) if k + 1 < _NDMA else None
        pending.wait()
        buf = bufs[k % 2]

        def body(i, carry):
            base = i * 64
            for u in range(4):
                w = buf[pl.ds(base + u * 16, 16)]
                lo = w & 0xFFFF
                hi = lax.shift_right_logical(w, 16)
                plsc.addupdate_scatter(hist, [lane_base + lo], ones)
                plsc.addupdate_scatter(hist, [hi_base + hi], ones)
            return carry

        lax.fori_loop(0, _CHUNK // 64, body, 0)
        pending = nxt
    pltpu.sync_copy(hist, out_hbm.at[wid])


def _sc_hist(b_words):
    mesh = plsc.VectorSubcoreMesh(core_axis_name="c", subcore_axis_name="s")
    f = pl.kernel(
        _sc_hist_body,
        mesh=mesh,
        out_type=jax.ShapeDtypeStruct((_NW, 2 * 16 * NBPAD), jnp.float32),
        scratch_types=[
            pltpu.VMEM((_CHUNK,), jnp.int32),
            pltpu.VMEM((_CHUNK,), jnp.int32),
            pltpu.VMEM((2 * 16 * NBPAD,), jnp.float32),
            pltpu.SemaphoreType.DMA,
            pltpu.SemaphoreType.DMA,
        ],
        compiler_params=pltpu.CompilerParams(needs_layout_passes=False),
    )
    return f(b_words)


def kernel(nlist, positions):
    t = jnp.transpose(nlist, (1, 2, 0))
    bins = _tc_bins(t)
    partials = _sc_hist(bins)
    hist = partials.reshape(_NW * 32, NBPAD).sum(0)
    shell_rs = jnp.linspace(R_MIN, R_MAX, 101)
    vols = shell_rs[1:] ** 3 - shell_rs[:-1] ** 3
    return hist[1:NB - 1] / vols
